# baseline (speedup 1.0000x reference)
import functools

import jax
import jax.numpy as jnp
from jax import lax
from jax.experimental import pallas as pl
from jax.experimental.pallas import tpu as pltpu
from jax.experimental.pallas import tpu_sc as plsc

B, H, W, C = 1, 112, 112, 384
N = B * H * W * C
OUT = N * 4
NC, NS, L = 2, 16, 16

CH = 1_605_632
NCHUNK = 12
ROUNDS = NCHUNK // NC

PER_TILE = N // NS
WINDOW = 2_048
NW = PER_TILE // WINDOW
CHS = CH // NS
NBUF = 3
NG = NW // NBUF

BLK = 128
KB = 17

_mesh = plsc.VectorSubcoreMesh(core_axis_name="c", subcore_axis_name="s")

_scratch = (
    [pltpu.VMEM((WINDOW,), jnp.int32) for _ in range(NBUF)]
    + [pltpu.VMEM((WINDOW,), jnp.bfloat16) for _ in range(NBUF)]
    + [pltpu.VMEM((KB, BLK), jnp.int32) for _ in range(NBUF)]
    + [pltpu.VMEM((KB, BLK), jnp.float32) for _ in range(NBUF)]
    + [pltpu.VMEM_SHARED((CH,), jnp.float32)]
    + [pltpu.SemaphoreType.DMA for _ in range(2 * NBUF)]
)


@functools.partial(
    pl.kernel,
    out_type=jax.ShapeDtypeStruct((OUT,), jnp.float32),
    mesh=_mesh,
    scratch_types=_scratch,
    compiler_params=pltpu.CompilerParams(needs_layout_passes=False),
)
def _unpool_scatter(idx_hbm, upd_hbm, zeros_hbm, out_hbm, *scratch):
    idx_bufs = scratch[:NBUF]
    upd_bufs = scratch[NBUF:2 * NBUF]
    cidx_bufs = scratch[2 * NBUF:3 * NBUF]
    cval_bufs = scratch[3 * NBUF:4 * NBUF]
    acc = scratch[4 * NBUF]
    sem_in = scratch[4 * NBUF + 1:4 * NBUF + 1 + NBUF]
    sem_fl = scratch[4 * NBUF + 1 + NBUF:]

    c = lax.axis_index("c")
    s = lax.axis_index("s")
    tile_in = s * PER_TILE
    acc_lo = s * CHS

    def in_copies(b, w):
        off = tile_in + w * WINDOW
        return (
            pltpu.make_async_copy(idx_hbm.at[pl.ds(off, WINDOW)],
                                  idx_bufs[b], sem_in[b]),
            pltpu.make_async_copy(upd_hbm.at[pl.ds(off, WINDOW)],
                                  upd_bufs[b], sem_in[b]),
        )

    def fire_in(b, w):
        for d in in_copies(b, w):
            d.start()

    def wait_in(b, w):
        for d in in_copies(b, w):
            d.wait()

    def fl_copy(b, j):
        return pltpu.make_async_copy(
            cval_bufs[b].at[j],
            acc.at[plsc.Indices(cidx_bufs[b].at[j], ignored_value=-1)],
            sem_fl[b],
        )

    def drain(b, npend):
        def body(_, carry):
            fl_copy(b, 0).wait()
            return carry
        lax.fori_loop(0, npend, body, 0)

    lanes = jnp.arange(L, dtype=jnp.int32)

    pltpu.sync_copy(zeros_hbm.at[pl.ds(0, CHS)], acc.at[pl.ds(acc_lo, CHS)])
    plsc.subcore_barrier()
    for b in range(NBUF):
        fire_in(b, b)

    pend = tuple(jnp.int32(0) for _ in range(NBUF))

    for r in range(ROUNDS):
        base = (NC * r + c) * CH

        def group_body(g, pend, base=base):
            pend = list(pend)
            for b in range(NBUF):
                w = g * NBUF + b

                drain(b, pend[b])
                wait_in(b, w)

                @plsc.parallel_loop(0, WINDOW, 2 * L, unroll=4,
                                    carry=jnp.full((L,), -1, jnp.int32))
                def ptr_final(ii, ptr, b=b):
                    uab = upd_bufs[b][pl.ds(ii, 2 * L)]
                    ua, ub = plsc.unpack(uab,
                                         format=plsc.PackFormat.INTERLEAVED)
                    for half, uval in ((0, ua), (1, ub)):
                        gidx = idx_bufs[b][pl.ds(ii + half * L, L)]
                        local = gidx - base
                        inb = plsc.bitcast(local, jnp.uint32) < jnp.uint32(CH)
                        pos = ptr + plsc.cumsum(jnp.where(inb, 1, 0))
                        row = lax.shift_right_logical(pos, 7)
                        col = lax.bitwise_and(pos, BLK - 1)
                        plsc.store_scatter(cidx_bufs[b], [row, col], local,
                                           mask=inb)
                        plsc.store_scatter(cval_bufs[b], [row, col], uval,
                                           mask=inb)
                        ptr = ptr + plsc.all_reduce_population_count(inb)
                    return ptr

                @pl.when(w + NBUF < NW)
                def _():
                    fire_in(b, w + NBUF)

                fill = jnp.max(ptr_final) + 1
                for k in range(L):
                    pos = fill + k * L + lanes
                    row = lax.shift_right_logical(pos, 7)
                    col = lax.bitwise_and(pos, BLK - 1)
                    plsc.store_scatter(cidx_bufs[b], [row, col],
                                       jnp.full((L,), -1, jnp.int32))

                nblk = lax.shift_right_logical(fill + BLK - 1, 7)

                def fire_blk(j, carry, b=b):
                    fl_copy(b, j).start(add=True)
                    return carry
                lax.fori_loop(0, nblk, fire_blk, 0)
                pend[b] = nblk
            return tuple(pend)

        pend = lax.fori_loop(0, NG, group_body, pend)

        for b in range(NBUF):
            drain(b, pend[b])
        pend = tuple(jnp.int32(0) for _ in range(NBUF))
        plsc.subcore_barrier()

        if r < ROUNDS - 1:
            for b in range(NBUF):
                fire_in(b, b)

        pltpu.sync_copy(acc.at[pl.ds(acc_lo, CHS)],
                        out_hbm.at[pl.ds(base + acc_lo, CHS)])

        if r < ROUNDS - 1:
            pltpu.sync_copy(zeros_hbm.at[pl.ds(0, CHS)],
                            acc.at[pl.ds(acc_lo, CHS)])
            plsc.subcore_barrier()


def kernel(updates, mask):
    idx = mask.reshape(-1)
    upd = (updates.reshape(-1, 2, L).transpose(0, 2, 1).reshape(-1)
           .astype(jnp.bfloat16))
    zeros = jnp.zeros((CHS,), jnp.float32)
    out = _unpool_scatter(idx, upd, zeros)
    return out.reshape(B, H * 2, W * 2, C)


# speedup vs baseline: 6.0431x; 6.0431x over previous
import functools

import jax
import jax.numpy as jnp
from jax import lax
from jax.experimental import pallas as pl
from jax.experimental.pallas import tpu as pltpu
from jax.experimental.pallas import tpu_sc as plsc

B, H, W, C = 1, 112, 112, 384
N = B * H * W * C
OUT = N * 4
NC, NS, L = 2, 16, 16

CH = 1_605_632
NCHUNK = 12
ROUNDS = NCHUNK // NC

PER_TILE = N // NS
WINDOW = 2_048
NW = PER_TILE // WINDOW
CHS = CH // NS
NBUF = 3
NG = NW // NBUF

BLK = 128
KB = 17

_mesh = plsc.VectorSubcoreMesh(core_axis_name="c", subcore_axis_name="s")

_scratch = (
    [pltpu.VMEM((WINDOW,), jnp.int32) for _ in range(NBUF)]
    + [pltpu.VMEM((WINDOW,), jnp.float32) for _ in range(NBUF)]
    + [pltpu.VMEM((KB, BLK), jnp.int32) for _ in range(NBUF)]
    + [pltpu.VMEM((KB, BLK), jnp.float32) for _ in range(NBUF)]
    + [pltpu.VMEM_SHARED((CH,), jnp.float32)]
    + [pltpu.SemaphoreType.DMA for _ in range(2 * NBUF)]
)


@functools.partial(
    pl.kernel,
    out_type=jax.ShapeDtypeStruct((OUT,), jnp.float32),
    mesh=_mesh,
    scratch_types=_scratch,
    compiler_params=pltpu.CompilerParams(needs_layout_passes=False),
)
def _unpool_scatter(idx_hbm, upd_hbm, zeros_hbm, out_hbm, *scratch):
    idx_bufs = scratch[:NBUF]
    upd_bufs = scratch[NBUF:2 * NBUF]
    cidx_bufs = scratch[2 * NBUF:3 * NBUF]
    cval_bufs = scratch[3 * NBUF:4 * NBUF]
    acc = scratch[4 * NBUF]
    sem_in = scratch[4 * NBUF + 1:4 * NBUF + 1 + NBUF]
    sem_fl = scratch[4 * NBUF + 1 + NBUF:]

    c = lax.axis_index("c")
    s = lax.axis_index("s")
    tile_in = s * PER_TILE
    acc_lo = s * CHS

    def in_copies(b, w):
        off = tile_in + w * WINDOW
        return (
            pltpu.make_async_copy(idx_hbm.at[pl.ds(off, WINDOW)],
                                  idx_bufs[b], sem_in[b]),
            pltpu.make_async_copy(upd_hbm.at[pl.ds(off, WINDOW)],
                                  upd_bufs[b], sem_in[b]),
        )

    def fire_in(b, w):
        for d in in_copies(b, w):
            d.start()

    def wait_in(b, w):
        for d in in_copies(b, w):
            d.wait()

    def fl_copy(b, j):
        return pltpu.make_async_copy(
            cval_bufs[b].at[j],
            acc.at[plsc.Indices(cidx_bufs[b].at[j], ignored_value=-1)],
            sem_fl[b],
        )

    def drain(b, npend):
        def body(_, carry):
            fl_copy(b, 0).wait()
            return carry
        lax.fori_loop(0, npend, body, 0)

    lanes = jnp.arange(L, dtype=jnp.int32)

    pltpu.sync_copy(zeros_hbm.at[pl.ds(0, CHS)], acc.at[pl.ds(acc_lo, CHS)])
    plsc.subcore_barrier()
    for b in range(NBUF):
        fire_in(b, b)

    pend = tuple(jnp.int32(0) for _ in range(NBUF))

    for r in range(ROUNDS):
        base = (NC * r + c) * CH

        def group_body(g, pend, base=base):
            pend = list(pend)
            for b in range(NBUF):
                w = g * NBUF + b

                drain(b, pend[b])
                wait_in(b, w)

                @plsc.parallel_loop(0, WINDOW, L, unroll=8,
                                    carry=jnp.full((L,), -1, jnp.int32))
                def ptr_final(ii, ptr, b=b):
                    gidx = idx_bufs[b][pl.ds(ii, L)]
                    local = gidx - base
                    inb = plsc.bitcast(local, jnp.uint32) < jnp.uint32(CH)
                    pos = ptr + plsc.cumsum(jnp.where(inb, 1, 0))
                    row = lax.shift_right_logical(pos, 7)
                    col = lax.bitwise_and(pos, BLK - 1)
                    plsc.store_scatter(cidx_bufs[b], [row, col], local,
                                       mask=inb)
                    plsc.store_scatter(cval_bufs[b], [row, col],
                                       upd_bufs[b][pl.ds(ii, L)], mask=inb)
                    return ptr + plsc.all_reduce_population_count(inb)

                @pl.when(w + NBUF < NW)
                def _():
                    fire_in(b, w + NBUF)

                fill = jnp.max(ptr_final) + 1
                for k in range(L):
                    pos = fill + k * L + lanes
                    row = lax.shift_right_logical(pos, 7)
                    col = lax.bitwise_and(pos, BLK - 1)
                    plsc.store_scatter(cidx_bufs[b], [row, col],
                                       jnp.full((L,), -1, jnp.int32))

                nblk = lax.shift_right_logical(fill + BLK - 1, 7)

                def fire_blk(j, carry, b=b):
                    fl_copy(b, j).start(add=True)
                    return carry
                lax.fori_loop(0, nblk, fire_blk, 0)
                pend[b] = nblk
            return tuple(pend)

        pend = lax.fori_loop(0, NG, group_body, pend)

        for b in range(NBUF):
            drain(b, pend[b])
        pend = tuple(jnp.int32(0) for _ in range(NBUF))
        plsc.subcore_barrier()

        if r < ROUNDS - 1:
            for b in range(NBUF):
                fire_in(b, b)

        if r < ROUNDS - 1:
            HALF = CHS // 2
            fl = [pltpu.make_async_copy(
                      acc.at[pl.ds(acc_lo + h * HALF, HALF)],
                      out_hbm.at[pl.ds(base + acc_lo + h * HALF, HALF)],
                      sem_fl[h]) for h in range(2)]
            zr = [pltpu.make_async_copy(
                      zeros_hbm.at[pl.ds(h * HALF, HALF)],
                      acc.at[pl.ds(acc_lo + h * HALF, HALF)],
                      sem_fl[h]) for h in range(2)]
            for h in range(2):
                fl[h].start()
            for h in range(2):
                fl[h].wait()
                zr[h].start()
            for h in range(2):
                zr[h].wait()
            plsc.subcore_barrier()
        else:
            pltpu.sync_copy(acc.at[pl.ds(acc_lo, CHS)],
                            out_hbm.at[pl.ds(base + acc_lo, CHS)])


def kernel(updates, mask):
    idx = mask.reshape(-1)
    upd = updates.reshape(-1)
    zeros = jnp.zeros((CHS,), jnp.float32)
    out = _unpool_scatter(idx, upd, zeros)
    return out.reshape(B, H * 2, W * 2, C)
